# baseline (speedup 1.0000x reference)
import dataclasses
import functools

import jax
import jax.numpy as jnp
from jax import lax
from jax.experimental import pallas as pl
from jax.experimental.pallas import tpu as pltpu
from jax.experimental.pallas import tpu_sc as plsc

_EPS = 1e-12
_NC, _NS = 2, 16
_NW = _NC * _NS
_LANES = 16


def _sc_gather(word_emb, x2, y2, h2, w2, ids_flat, bbox_t):
    tok = ids_flat.shape[0]
    hdim = word_emb.shape[1]
    hw = hdim // 2
    b_per_w = tok // _NW
    ch = 64
    n_chunks = b_per_w // ch
    assert tok % _NW == 0 and b_per_w % ch == 0 and hdim % (4 * _LANES) == 0

    mesh = plsc.VectorSubcoreMesh(
        core_axis_name="c", subcore_axis_name="s",
        num_cores=_NC, num_subcores=_NS)

    @functools.partial(
        pl.kernel,
        out_type=[
            jax.ShapeDtypeStruct((tok, hdim), jnp.float32),
            jax.ShapeDtypeStruct((tok, hdim), jnp.bfloat16),
        ],
        mesh=mesh,
        scratch_types=[
            pltpu.VMEM((ch,), jnp.int32),
            pltpu.VMEM((6, ch), jnp.int32),
            pltpu.VMEM((ch, hdim), jnp.float32),
            pltpu.VMEM((ch, hdim), jnp.bfloat16),
            pltpu.VMEM((ch, hw), jnp.int32),
            pltpu.VMEM((ch, hw), jnp.int32),
            pltpu.SemaphoreType.DMA,
            pltpu.SemaphoreType.DMA,
            pltpu.SemaphoreType.DMA,
            pltpu.SemaphoreType.DMA,
        ],
        compiler_params=dataclasses.replace(
            pltpu.CompilerParams(), needs_layout_passes=False),
    )
    def k(word_hbm, x_hbm, y_hbm, h_hbm, w_hbm, ids_hbm, bbox_hbm,
          outw_hbm, outs_hbm,
          widx, bidx, wbuf, acc, buf_a, buf_b,
          sem_w, sem_a, sem_b, sem_c):
        sid = lax.axis_index("s")
        wid = sid * _NC + lax.axis_index("c")

        @pl.loop(0, n_chunks)
        def _chunk(c):
            base = wid * b_per_w + c * ch
            pltpu.sync_copy(ids_hbm.at[pl.ds(base, ch)], widx)
            for j in range(4):
                pltpu.sync_copy(bbox_hbm.at[j, pl.ds(base, ch)], bidx.at[j])
            for i in range(0, ch, _LANES):
                s = pl.ds(i, _LANES)
                bidx[4, s] = bidx[3, s] - bidx[1, s]
                bidx[5, s] = bidx[2, s] - bidx[0, s]

            cp_w = pltpu.async_copy(word_hbm.at[widx], wbuf, sem_w)

            tabs = (x_hbm, y_hbm, x_hbm, h_hbm, w_hbm, y_hbm)
            irow = (0, 1, 2, 4, 5, 3)
            bufs = (buf_a, buf_b, buf_a, buf_b, buf_a, buf_b)
            sems = (sem_a, sem_b, sem_a, sem_b, sem_a, sem_b)
            cps = [None] * 6
            cps[0] = pltpu.async_copy(
                tabs[0].at[bidx.at[irow[0]]], bufs[0], sems[0])
            cps[1] = pltpu.async_copy(
                tabs[1].at[bidx.at[irow[1]]], bufs[1], sems[1])
            for j in range(6):
                cps[j].wait()
                if j + 2 < 6:
                    cps[j + 2] = pltpu.async_copy(
                        tabs[j + 2].at[bidx.at[irow[j + 2]]],
                        bufs[j + 2], sems[j + 2])
                buf = bufs[j]

                if j == 0:
                    @pl.loop(0, ch)
                    def _row0(r):
                        for i in range(0, hw, _LANES):
                            acc[r, pl.ds(2 * i, 2 * _LANES)] = plsc.bitcast(
                                buf[r, pl.ds(i, _LANES)], jnp.bfloat16)
                else:
                    @pl.loop(0, ch)
                    def _row(r):
                        for i in range(0, hw, _LANES):
                            plsc.addupdate(
                                acc.at[r, pl.ds(2 * i, 2 * _LANES)],
                                plsc.bitcast(buf[r, pl.ds(i, _LANES)],
                                             jnp.bfloat16))

            cp_w.wait()
            pltpu.sync_copy(wbuf, outw_hbm.at[pl.ds(base, ch)])
            pltpu.sync_copy(acc, outs_hbm.at[pl.ds(base, ch)])

    return k(word_emb, x2, y2, h2, w2, ids_flat, bbox_t)


def _tc_finish(words, small, pos_emb, tids2, tt_pad, gamma2, beta2):
    tok, hdim = words.shape
    blk = 256
    n = tok // blk
    s_len = pos_emb.shape[0]
    pos_blocks = s_len // blk

    def body(g_ref, sm_ref, pos_ref, tid_ref, ttab_ref, gam_ref, bet_ref,
             o_ref):
        x = g_ref[...] + sm_ref[...].astype(jnp.float32) + pos_ref[...]
        tid = tid_ref[...]
        x = x + jnp.where(tid < 1, ttab_ref[0:1, :], ttab_ref[1:2, :])
        mean = jnp.mean(x, axis=-1, keepdims=True)
        xc = x - mean
        var = jnp.mean(xc * xc, axis=-1, keepdims=True)
        o_ref[...] = xc * lax.rsqrt(var + _EPS) * gam_ref[...] + bet_ref[...]

    return pl.pallas_call(
        body,
        grid=(n,),
        in_specs=[
            pl.BlockSpec((blk, hdim), lambda i: (i, 0)),
            pl.BlockSpec((blk, hdim), lambda i: (i, 0)),
            pl.BlockSpec((blk, hdim), lambda i: (i % pos_blocks, 0)),
            pl.BlockSpec((blk, 1), lambda i: (i, 0)),
            pl.BlockSpec((8, hdim), lambda i: (0, 0)),
            pl.BlockSpec((1, hdim), lambda i: (0, 0)),
            pl.BlockSpec((1, hdim), lambda i: (0, 0)),
        ],
        out_specs=pl.BlockSpec((blk, hdim), lambda i: (i, 0)),
        out_shape=jax.ShapeDtypeStruct((tok, hdim), jnp.float32),
    )(words, small, pos_emb, tids2, tt_pad, gamma2, beta2)


def _packed2(tab):
    rows, hdim = tab.shape
    b = tab.astype(jnp.bfloat16).reshape(rows, hdim // 2, 2)
    return lax.bitcast_convert_type(b, jnp.int32)


def kernel(input_ids, bbox, token_type_ids, word_emb, pos_emb,
           x_emb, y_emb, h_emb, w_emb, tt_emb, ln_gamma, ln_beta):
    b, s = input_ids.shape
    hdim = word_emb.shape[1]
    tok = b * s

    ids_flat = input_ids.reshape(tok)
    bbox_t = bbox.reshape(tok, 4).T

    words, small = _sc_gather(word_emb, _packed2(x_emb), _packed2(y_emb),
                              _packed2(h_emb), _packed2(w_emb),
                              ids_flat, bbox_t)

    tids2 = token_type_ids.reshape(tok, 1)
    tt_pad = jnp.zeros((8, hdim), tt_emb.dtype).at[:2, :].set(tt_emb)
    out = _tc_finish(words, small, pos_emb, tids2,
                     tt_pad, ln_gamma.reshape(1, hdim),
                     ln_beta.reshape(1, hdim))
    return out.reshape(b, s, hdim)


# speedup vs baseline: 1.5655x; 1.5655x over previous
import functools

import jax
import jax.numpy as jnp
from jax import lax
from jax.experimental import pallas as pl
from jax.experimental.pallas import tpu as pltpu
from jax.experimental.pallas import tpu_sc as plsc

_EPS = 1e-12
_NC, _NS = 2, 16
_NW = _NC * _NS
_LANES = 16


_CH = 32


def _sc_gather_sum(word_emb, x_emb, y_emb, h_emb, w_emb, idx5c, tok):
    hdim = word_emb.shape[1]
    b_per_w = tok // _NW
    ch = _CH
    n_chunks = b_per_w // ch
    assert tok % _NW == 0 and b_per_w % (2 * ch) == 0 and hdim % _LANES == 0

    mesh = plsc.VectorSubcoreMesh(
        core_axis_name="c", subcore_axis_name="s",
        num_cores=_NC, num_subcores=_NS)

    @functools.partial(
        pl.kernel,
        out_type=jax.ShapeDtypeStruct((tok, hdim), jnp.float32),
        mesh=mesh,
        scratch_types=[
            pltpu.VMEM((8 * ch,), jnp.int32),
            pltpu.VMEM((8 * ch,), jnp.int32),
            pltpu.VMEM((ch, hdim), jnp.float32),
            pltpu.VMEM((ch, hdim), jnp.float32),
            pltpu.VMEM((ch, hdim), jnp.float32),
            pltpu.VMEM((ch, hdim), jnp.float32),
            pltpu.SemaphoreType.DMA,
            pltpu.SemaphoreType.DMA,
            pltpu.SemaphoreType.DMA,
            pltpu.SemaphoreType.DMA,
            pltpu.SemaphoreType.DMA,
        ],
    )
    def k(word_hbm, x_hbm, y_hbm, h_hbm, w_hbm, idx5_hbm, out_hbm,
          idx0, idx1, acc0, acc1, buf_a, buf_b,
          sem_i, sem_w, sem_a, sem_b, sem_o):
        wid = lax.axis_index("s") * _NC + lax.axis_index("c")
        w0 = wid * b_per_w
        t0 = wid * n_chunks

        def fetch_idx(c, idxb):
            return pltpu.async_copy(idx5_hbm.at[t0 + c], idxb, sem_i)

        fetch_idx(0, idx0)

        def chunk_body(c, idxb, other_idxb, acc, last_parity):
            base = w0 + c * ch

            def grp(j):
                return idxb.at[pl.ds(j * ch, ch)]

            pltpu.make_async_copy(idx5_hbm.at[t0 + c], idxb, sem_i).wait()
            if not last_parity:
                fetch_idx(c + 1, other_idxb)
            else:
                @pl.when(c + 1 < n_chunks)
                def _():
                    fetch_idx(c + 1, other_idxb)
            for i in range(0, ch, _LANES):
                idxb[pl.ds(5 * ch + i, _LANES)] = (
                    idxb[pl.ds(4 * ch + i, _LANES)]
                    - idxb[pl.ds(2 * ch + i, _LANES)])
                idxb[pl.ds(6 * ch + i, _LANES)] = (
                    idxb[pl.ds(3 * ch + i, _LANES)]
                    - idxb[pl.ds(1 * ch + i, _LANES)])

            tabs = (x_hbm, y_hbm, x_hbm, h_hbm, w_hbm, y_hbm)
            irow = (1, 2, 3, 5, 6, 4)
            bufs = (buf_a, buf_b, buf_a, buf_b, buf_a, buf_b)
            sems = (sem_a, sem_b, sem_a, sem_b, sem_a, sem_b)
            cps = [None] * 6
            cps[0] = pltpu.async_copy(
                tabs[0].at[grp(irow[0])], bufs[0], sems[0])
            cps[1] = pltpu.async_copy(
                tabs[1].at[grp(irow[1])], bufs[1], sems[1])

            @pl.when(c >= 2)
            def _():
                pltpu.make_async_copy(
                    acc, out_hbm.at[pl.ds(base, ch)], sem_o).wait()
            cp_w = pltpu.async_copy(word_hbm.at[grp(0)], acc, sem_w)

            cp_w.wait()
            for j in range(6):
                cps[j].wait()
                if j + 2 < 6:
                    cps[j + 2] = pltpu.async_copy(
                        tabs[j + 2].at[grp(irow[j + 2])],
                        bufs[j + 2], sems[j + 2])
                buf = bufs[j]

                @pl.loop(0, ch)
                def _row(r):
                    for i in range(0, hdim, _LANES):
                        s = pl.ds(i, _LANES)
                        plsc.addupdate(acc.at[r, s], buf[r, s])

            pltpu.async_copy(acc, out_hbm.at[pl.ds(base, ch)], sem_o)

        @pl.loop(0, n_chunks, step=2)
        def _chunks(c):
            chunk_body(c, idx0, idx1, acc0, False)
            chunk_body(c + 1, idx1, idx0, acc1, True)

        for acc in (acc0, acc1):
            pltpu.make_async_copy(
                acc, out_hbm.at[pl.ds(w0, ch)], sem_o).wait()

    return k(word_emb, x_emb, y_emb, h_emb, w_emb, idx5c)


def _tc_finish(gsum, pos_emb, tids2, tt_pad, gamma2, beta2):
    tok, hdim = gsum.shape
    blk = 256
    n = tok // blk
    s_len = pos_emb.shape[0]
    pos_blocks = s_len // blk

    def body(g_ref, pos_ref, tid_ref, ttab_ref, gam_ref, bet_ref, o_ref):
        x = g_ref[...] + pos_ref[...]
        tid = tid_ref[...]
        x = x + jnp.where(tid < 1, ttab_ref[0:1, :], ttab_ref[1:2, :])
        mean = jnp.mean(x, axis=-1, keepdims=True)
        xc = x - mean
        var = jnp.mean(xc * xc, axis=-1, keepdims=True)
        o_ref[...] = xc * lax.rsqrt(var + _EPS) * gam_ref[...] + bet_ref[...]

    return pl.pallas_call(
        body,
        grid=(n,),
        in_specs=[
            pl.BlockSpec((blk, hdim), lambda i: (i, 0)),
            pl.BlockSpec((blk, hdim), lambda i: (i % pos_blocks, 0)),
            pl.BlockSpec((blk, 1), lambda i: (i, 0)),
            pl.BlockSpec((8, hdim), lambda i: (0, 0)),
            pl.BlockSpec((1, hdim), lambda i: (0, 0)),
            pl.BlockSpec((1, hdim), lambda i: (0, 0)),
        ],
        out_specs=pl.BlockSpec((blk, hdim), lambda i: (i, 0)),
        out_shape=jax.ShapeDtypeStruct((tok, hdim), jnp.float32),
    )(gsum, pos_emb, tids2, tt_pad, gamma2, beta2)


def kernel(input_ids, bbox, token_type_ids, word_emb, pos_emb,
           x_emb, y_emb, h_emb, w_emb, tt_emb, ln_gamma, ln_beta):
    b, s = input_ids.shape
    hdim = word_emb.shape[1]
    tok = b * s

    ids_flat = input_ids.reshape(tok)
    bbox_t = bbox.reshape(tok, 4).T
    nck = tok // _CH
    idx5c = (jnp.concatenate([ids_flat[None, :], bbox_t], axis=0)
             .reshape(5, nck, _CH).transpose(1, 0, 2))
    idx5c = jnp.concatenate(
        [idx5c, jnp.zeros((nck, 3, _CH), jnp.int32)], axis=1)
    idx5c = idx5c.reshape(nck, 8 * _CH)

    gsum = _sc_gather_sum(word_emb, x_emb, y_emb, h_emb, w_emb, idx5c, tok)

    tids2 = token_type_ids.reshape(tok, 1)
    tt_pad = jnp.zeros((8, hdim), tt_emb.dtype).at[:2, :].set(tt_emb)
    out = _tc_finish(gsum, pos_emb, tids2, tt_pad,
                     ln_gamma.reshape(1, hdim), ln_beta.reshape(1, hdim))
    return out.reshape(b, s, hdim)
